# Initial kernel scaffold; baseline (speedup 1.0000x reference)
import jax
import jax.numpy as jnp
from jax.experimental import pallas as pl


def kernel(x, w1, b1, w2, b2):
    raise NotImplementedError("write your pallas kernel here")


# speedup vs baseline: 1.2080x; 1.2080x over previous
import functools

import jax
import jax.numpy as jnp
from jax.experimental import pallas as pl
from jax.experimental.pallas import tpu as pltpu

_LANE = 128


def _se_step(x_ref, w1t_ref, b1_ref, w2t_ref, b2_ref, o_ref, *, inv_hw):
    x = x_ref[...]
    s = jnp.sum(x, axis=-1, dtype=jnp.float32) * jnp.float32(inv_hw)
    h = jnp.dot(s, w1t_ref[...], preferred_element_type=jnp.float32) + b1_ref[...]
    h = h * jax.nn.sigmoid(h)
    g = jnp.dot(h, w2t_ref[...], preferred_element_type=jnp.float32) + b2_ref[...]
    g = jax.nn.sigmoid(g)
    o_ref[...] = x * g[:, :, None]


def kernel(x, w1, b1, w2, b2):
    N, C, H, W = x.shape
    R = w1.shape[0]
    HW = H * W
    HWp = ((HW + _LANE - 1) // _LANE) * _LANE

    x_flat = x.reshape(N, C, HW)
    if HWp != HW:
        x_flat = jnp.pad(x_flat, ((0, 0), (0, 0), (0, HWp - HW)))

    itemsize = jnp.dtype(x.dtype).itemsize
    per_sample = C * HWp * itemsize
    nb = 1
    for d in range(1, N + 1):
        if N % d == 0 and N // d >= 4 and 4 * d * per_sample <= (48 << 20):
            nb = d

    out_flat = pl.pallas_call(
        functools.partial(_se_step, inv_hw=1.0 / HW),
        out_shape=jax.ShapeDtypeStruct((N, C, HWp), x.dtype),
        grid=(N // nb,),
        in_specs=[
            pl.BlockSpec((nb, C, HWp), lambda i: (i, 0, 0)),
            pl.BlockSpec((C, R), lambda i: (0, 0)),
            pl.BlockSpec((1, R), lambda i: (0, 0)),
            pl.BlockSpec((R, C), lambda i: (0, 0)),
            pl.BlockSpec((1, C), lambda i: (0, 0)),
        ],
        out_specs=pl.BlockSpec((nb, C, HWp), lambda i: (i, 0, 0)),
        compiler_params=pltpu.CompilerParams(
            dimension_semantics=("parallel",),
            vmem_limit_bytes=int(56 << 20)),
    )(x_flat,
      w1.T.astype(jnp.float32),
      b1.reshape(1, R).astype(jnp.float32),
      w2.T.astype(jnp.float32),
      b2.reshape(1, C).astype(jnp.float32))

    if HWp != HW:
        out_flat = out_flat[:, :, :HW]
    return out_flat.reshape(N, C, H, W)
